# baseline (speedup 1.0000x reference)
import functools
import math

import jax
import jax.numpy as jnp
from jax import lax
from jax.experimental import pallas as pl
from jax.experimental.pallas import tpu as pltpu, tpu_sc as plsc

N_FEATURES = 32
SCALE = math.sqrt(N_FEATURES)

NUM_CORES = 2
NUM_SUBCORES = 16
NUM_WORKERS = NUM_CORES * NUM_SUBCORES

LANES = 16
IDX_PER_STREAM = 128
STREAMS_PER_CHUNK = 4
CHUNK = IDX_PER_STREAM * STREAMS_PER_CHUNK
FB = N_FEATURES // 8
IBL = CHUNK // IDX_PER_STREAM


def _emb_kernel(n_chunks, x_hbm, lut_hbm, out_hbm,
                idx0, idx1, rows0, rows1, strips0, strips1,
                gsem0, gsem1, wsem0, wsem1):
    wid = lax.axis_index("s") * NUM_CORES + lax.axis_index("c")
    chunk_base = wid * n_chunks
    groups_per_j = x_hbm.shape[1] // IBL
    idx = (idx0, idx1)
    rows = (rows0, rows1)
    strips = (strips0, strips1)
    gsem = (gsem0, gsem1)
    wsem = (wsem0, wsem1)
    iota = lax.iota(jnp.int32, LANES)

    def coords(c):
        return c // groups_per_j, c % groups_per_j

    def fire(c, s):
        j, g = coords(c)
        pltpu.sync_copy(x_hbm.at[j, pl.ds(g * IBL, IBL)], idx[s])
        for k in range(STREAMS_PER_CHUNK):
            pltpu.async_copy(
                lut_hbm.at[idx[s].at[k]],
                rows[s].at[pl.ds(k * IDX_PER_STREAM, IDX_PER_STREAM)],
                gsem[s],
            )

    def drain_gathers(s):
        for k in range(STREAMS_PER_CHUNK):
            pltpu.make_async_copy(
                lut_hbm.at[idx[s].at[k]],
                rows[s].at[pl.ds(k * IDX_PER_STREAM, IDX_PER_STREAM)],
                gsem[s],
            ).wait()

    def drain_write(s):
        for fb in range(FB):
            pltpu.make_async_copy(
                strips[s].at[fb], out_hbm.at[0, fb, pl.ds(0, IBL)], wsem[s]
            ).wait()

    def process(c, s):
        j, g = coords(c)
        drain_gathers(s)

        @pl.loop(0, CHUNK // LANES)
        def _tr(t16):
            ibl = t16 // (IDX_PER_STREAM // LANES)
            il16 = t16 % (IDX_PER_STREAM // LANES)
            row_ids = t16 * LANES + iota
            for fb in range(FB):
                for fs in range(8):
                    col_ids = jnp.full((LANES,), fb * 8 + fs, jnp.int32)
                    v = plsc.load_gather(rows[s], [row_ids, col_ids])
                    strips[s][fb, ibl, fs, pl.ds(il16 * LANES, LANES)] = (
                        v * SCALE
                    )

        for fb in range(FB):
            pltpu.async_copy(
                strips[s].at[fb], out_hbm.at[j, fb, pl.ds(g * IBL, IBL)],
                wsem[s],
            )

    fire(chunk_base, 0)

    @pl.loop(0, n_chunks // 2)
    def _pair(gi):
        a = chunk_base + 2 * gi

        @pl.when(gi > 0)
        def _():
            drain_write(1)

        fire(a + 1, 1)
        process(a, 0)

        @pl.when(2 * gi + 2 < n_chunks)
        def _():
            drain_write(0)
            fire(a + 2, 0)

        process(a + 1, 1)

    drain_write(0)
    drain_write(1)


@jax.jit
def _embedding_lookup(x3d, lut):
    n_j, n_ib = x3d.shape[0], x3d.shape[1]
    b_total = n_j * n_ib * x3d.shape[2]
    n_chunks = b_total // (NUM_WORKERS * CHUNK)
    mesh = plsc.VectorSubcoreMesh(
        core_axis_name="c", subcore_axis_name="s",
        num_cores=NUM_CORES, num_subcores=NUM_SUBCORES,
    )
    run = pl.kernel(
        functools.partial(_emb_kernel, n_chunks),
        out_type=jax.ShapeDtypeStruct(
            (n_j, FB, n_ib, 8, IDX_PER_STREAM), jnp.float32
        ),
        mesh=mesh,
        scratch_types=[
            pltpu.VMEM((STREAMS_PER_CHUNK, IDX_PER_STREAM), jnp.int32),
            pltpu.VMEM((STREAMS_PER_CHUNK, IDX_PER_STREAM), jnp.int32),
            pltpu.VMEM((CHUNK, N_FEATURES), jnp.float32),
            pltpu.VMEM((CHUNK, N_FEATURES), jnp.float32),
            pltpu.VMEM((FB, IBL, 8, IDX_PER_STREAM), jnp.float32),
            pltpu.VMEM((FB, IBL, 8, IDX_PER_STREAM), jnp.float32),
            pltpu.SemaphoreType.DMA,
            pltpu.SemaphoreType.DMA,
            pltpu.SemaphoreType.DMA,
            pltpu.SemaphoreType.DMA,
        ],
        compiler_params=pltpu.CompilerParams(
            use_tc_tiling_on_sc=False, needs_layout_passes=False
        ),
    )
    return run(x3d, lut)


def kernel(x, lut):
    n_tok, n_j = x.shape
    x3d = jnp.transpose(x).reshape(n_j, n_tok // IDX_PER_STREAM,
                                   IDX_PER_STREAM)
    out_phys = _embedding_lookup(x3d, lut)
    out = jnp.transpose(out_phys, (2, 4, 0, 1, 3)).reshape(
        n_tok, n_j, N_FEATURES
    )
    return out


# speedup vs baseline: 1.4379x; 1.4379x over previous
import functools
import math

import jax
import jax.numpy as jnp
from jax import lax
from jax.experimental import pallas as pl
from jax.experimental.pallas import tpu as pltpu, tpu_sc as plsc

N_FEATURES = 32
SCALE = math.sqrt(N_FEATURES)

NUM_CORES = 2
NUM_SUBCORES = 16
NUM_WORKERS = NUM_CORES * NUM_SUBCORES

LANES = 16
IDX_PER_STREAM = 128
STREAMS_PER_CHUNK = 4
CHUNK = IDX_PER_STREAM * STREAMS_PER_CHUNK
FB = N_FEATURES // 8
IBL = CHUNK // IDX_PER_STREAM


def _emb_kernel(n_chunks, x_hbm, lut_hbm, out_hbm,
                idx0, idx1, rows0, rows1, strips0, strips1,
                gsem0, gsem1, wsem0, wsem1):
    wid = lax.axis_index("s") * NUM_CORES + lax.axis_index("c")
    chunk_base = wid * n_chunks
    groups_per_j = x_hbm.shape[1] // IBL
    idx = (idx0, idx1)
    rows = (rows0, rows1)
    strips = (strips0, strips1)
    gsem = (gsem0, gsem1)
    wsem = (wsem0, wsem1)
    iota = lax.iota(jnp.int32, LANES)

    def coords(c):
        return c // groups_per_j, c % groups_per_j

    def fire(c, s):
        j, g = coords(c)
        pltpu.sync_copy(x_hbm.at[j, pl.ds(g * IBL, IBL)], idx[s])
        for k in range(STREAMS_PER_CHUNK):
            pltpu.async_copy(
                lut_hbm.at[idx[s].at[k]],
                rows[s].at[pl.ds(k * IDX_PER_STREAM, IDX_PER_STREAM)],
                gsem[s],
            )

    def drain_gathers(s):
        for k in range(STREAMS_PER_CHUNK):
            pltpu.make_async_copy(
                lut_hbm.at[idx[s].at[k]],
                rows[s].at[pl.ds(k * IDX_PER_STREAM, IDX_PER_STREAM)],
                gsem[s],
            ).wait()

    def drain_write(s):
        for fb in range(FB):
            pltpu.make_async_copy(
                strips[s].at[fb], out_hbm.at[0, fb, pl.ds(0, IBL)], wsem[s]
            ).wait()

    def process(c, s):
        j, g = coords(c)
        drain_gathers(s)

        col_ids = [jnp.full((LANES,), f, jnp.int32)
                   for f in range(N_FEATURES)]

        @plsc.parallel_loop(0, CHUNK // LANES, unroll=2)
        def _tr(t16):
            ibl = t16 // (IDX_PER_STREAM // LANES)
            il16 = t16 % (IDX_PER_STREAM // LANES)
            row_ids = t16 * LANES + iota
            for fb in range(FB):
                for fs in range(8):
                    v = plsc.load_gather(rows[s], [row_ids, col_ids[fb * 8 + fs]])
                    strips[s][fb, ibl, fs, pl.ds(il16 * LANES, LANES)] = (
                        v * SCALE
                    )

        for fb in range(FB):
            pltpu.async_copy(
                strips[s].at[fb], out_hbm.at[j, fb, pl.ds(g * IBL, IBL)],
                wsem[s],
            )

    fire(chunk_base, 0)

    @pl.loop(0, n_chunks // 2)
    def _pair(gi):
        a = chunk_base + 2 * gi

        @pl.when(gi > 0)
        def _():
            drain_write(1)

        fire(a + 1, 1)
        process(a, 0)

        @pl.when(2 * gi + 2 < n_chunks)
        def _():
            drain_write(0)
            fire(a + 2, 0)

        process(a + 1, 1)

    drain_write(0)
    drain_write(1)


@jax.jit
def _embedding_lookup(x3d, lut):
    n_j, n_ib = x3d.shape[0], x3d.shape[1]
    b_total = n_j * n_ib * x3d.shape[2]
    n_chunks = b_total // (NUM_WORKERS * CHUNK)
    mesh = plsc.VectorSubcoreMesh(
        core_axis_name="c", subcore_axis_name="s",
        num_cores=NUM_CORES, num_subcores=NUM_SUBCORES,
    )
    run = pl.kernel(
        functools.partial(_emb_kernel, n_chunks),
        out_type=jax.ShapeDtypeStruct(
            (n_j, FB, n_ib, 8, IDX_PER_STREAM), jnp.float32
        ),
        mesh=mesh,
        scratch_types=[
            pltpu.VMEM((STREAMS_PER_CHUNK, IDX_PER_STREAM), jnp.int32),
            pltpu.VMEM((STREAMS_PER_CHUNK, IDX_PER_STREAM), jnp.int32),
            pltpu.VMEM((CHUNK, N_FEATURES), jnp.float32),
            pltpu.VMEM((CHUNK, N_FEATURES), jnp.float32),
            pltpu.VMEM((FB, IBL, 8, IDX_PER_STREAM), jnp.float32),
            pltpu.VMEM((FB, IBL, 8, IDX_PER_STREAM), jnp.float32),
            pltpu.SemaphoreType.DMA,
            pltpu.SemaphoreType.DMA,
            pltpu.SemaphoreType.DMA,
            pltpu.SemaphoreType.DMA,
        ],
        compiler_params=pltpu.CompilerParams(
            use_tc_tiling_on_sc=False, needs_layout_passes=False
        ),
    )
    return run(x3d, lut)


def kernel(x, lut):
    n_tok, n_j = x.shape
    x3d = jnp.transpose(x).reshape(n_j, n_tok // IDX_PER_STREAM,
                                   IDX_PER_STREAM)
    out_phys = _embedding_lookup(x3d, lut)
    out = jnp.transpose(out_phys, (2, 4, 0, 1, 3)).reshape(
        n_tok, n_j, N_FEATURES
    )
    return out


# speedup vs baseline: 1.4688x; 1.0214x over previous
import functools
import math

import jax
import jax.numpy as jnp
from jax import lax
from jax.experimental import pallas as pl
from jax.experimental.pallas import tpu as pltpu, tpu_sc as plsc

N_FEATURES = 32
SCALE = math.sqrt(N_FEATURES)

NUM_CORES = 2
NUM_SUBCORES = 16
NUM_WORKERS = NUM_CORES * NUM_SUBCORES

LANES = 16
IDX_PER_STREAM = 128
STREAMS_PER_CHUNK = 4
CHUNK = IDX_PER_STREAM * STREAMS_PER_CHUNK
FB = N_FEATURES // 8
IBL = CHUNK // IDX_PER_STREAM


def _emb_kernel(n_chunks, x_hbm, lut_hbm, out_hbm,
                idx0, idx1, rows0, rows1, strips0, strips1,
                gsem0, gsem1, wsem0, wsem1):
    wid = lax.axis_index("s") * NUM_CORES + lax.axis_index("c")
    chunk_base = wid * n_chunks
    groups_per_j = x_hbm.shape[1] // IBL
    idx = (idx0, idx1)
    rows = (rows0, rows1)
    strips = (strips0, strips1)
    gsem = (gsem0, gsem1)
    wsem = (wsem0, wsem1)
    iota = lax.iota(jnp.int32, LANES)

    def coords(c):
        return c // groups_per_j, c % groups_per_j

    def fire(c, s):
        j, g = coords(c)
        pltpu.sync_copy(x_hbm.at[j, pl.ds(g * IBL, IBL)], idx[s])
        for k in range(STREAMS_PER_CHUNK):
            pltpu.async_copy(
                lut_hbm.at[idx[s].at[k]],
                rows[s].at[pl.ds(k * IDX_PER_STREAM, IDX_PER_STREAM)],
                gsem[s],
            )

    def drain_gathers(s):
        for k in range(STREAMS_PER_CHUNK):
            pltpu.make_async_copy(
                lut_hbm.at[idx[s].at[k]],
                rows[s].at[pl.ds(k * IDX_PER_STREAM, IDX_PER_STREAM)],
                gsem[s],
            ).wait()

    def drain_write(s):
        for fb in range(FB):
            pltpu.make_async_copy(
                strips[s].at[fb], out_hbm.at[0, fb, pl.ds(0, IBL)], wsem[s]
            ).wait()

    def process(c, s):
        j, g = coords(c)
        drain_gathers(s)

        col_ids = [jnp.full((LANES,), f, jnp.int32)
                   for f in range(N_FEATURES)]

        @plsc.parallel_loop(0, CHUNK // LANES, unroll=4)
        def _tr(t16):
            ibl = t16 // (IDX_PER_STREAM // LANES)
            il16 = t16 % (IDX_PER_STREAM // LANES)
            row_ids = t16 * LANES + iota
            for fb in range(FB):
                for fs in range(8):
                    v = plsc.load_gather(rows[s], [row_ids, col_ids[fb * 8 + fs]])
                    strips[s][fb, ibl, fs, pl.ds(il16 * LANES, LANES)] = (
                        v * SCALE
                    )

        for fb in range(FB):
            pltpu.async_copy(
                strips[s].at[fb], out_hbm.at[j, fb, pl.ds(g * IBL, IBL)],
                wsem[s],
            )

    fire(chunk_base, 0)

    @pl.loop(0, n_chunks // 2)
    def _pair(gi):
        a = chunk_base + 2 * gi

        @pl.when(gi > 0)
        def _():
            drain_write(1)

        fire(a + 1, 1)
        process(a, 0)

        @pl.when(2 * gi + 2 < n_chunks)
        def _():
            drain_write(0)
            fire(a + 2, 0)

        process(a + 1, 1)

    drain_write(0)
    drain_write(1)


@jax.jit
def _embedding_lookup(x3d, lut):
    n_j, n_ib = x3d.shape[0], x3d.shape[1]
    b_total = n_j * n_ib * x3d.shape[2]
    n_chunks = b_total // (NUM_WORKERS * CHUNK)
    mesh = plsc.VectorSubcoreMesh(
        core_axis_name="c", subcore_axis_name="s",
        num_cores=NUM_CORES, num_subcores=NUM_SUBCORES,
    )
    run = pl.kernel(
        functools.partial(_emb_kernel, n_chunks),
        out_type=jax.ShapeDtypeStruct(
            (n_j, FB, n_ib, 8, IDX_PER_STREAM), jnp.float32
        ),
        mesh=mesh,
        scratch_types=[
            pltpu.VMEM((STREAMS_PER_CHUNK, IDX_PER_STREAM), jnp.int32),
            pltpu.VMEM((STREAMS_PER_CHUNK, IDX_PER_STREAM), jnp.int32),
            pltpu.VMEM((CHUNK, N_FEATURES), jnp.float32),
            pltpu.VMEM((CHUNK, N_FEATURES), jnp.float32),
            pltpu.VMEM((FB, IBL, 8, IDX_PER_STREAM), jnp.float32),
            pltpu.VMEM((FB, IBL, 8, IDX_PER_STREAM), jnp.float32),
            pltpu.SemaphoreType.DMA,
            pltpu.SemaphoreType.DMA,
            pltpu.SemaphoreType.DMA,
            pltpu.SemaphoreType.DMA,
        ],
        compiler_params=pltpu.CompilerParams(
            use_tc_tiling_on_sc=False, needs_layout_passes=False
        ),
    )
    return run(x3d, lut)


def kernel(x, lut):
    n_tok, n_j = x.shape
    x3d = jnp.transpose(x).reshape(n_j, n_tok // IDX_PER_STREAM,
                                   IDX_PER_STREAM)
    out_phys = _embedding_lookup(x3d, lut)
    out = jnp.transpose(out_phys, (2, 4, 0, 1, 3)).reshape(
        n_tok, n_j, N_FEATURES
    )
    return out
